# baseline (speedup 1.0000x reference)
import functools

import jax
import jax.numpy as jnp
from jax import lax
from jax.experimental import pallas as pl
from jax.experimental.pallas import tpu as pltpu
from jax.experimental.pallas import tpu_sc as plsc

N = 50000
E = 800000
H = 64
HH = 32
CH = 128
NCHUNK = E // CH
NCHP = 6272
EP = NCHP * CH
CPT = NCHP // 16
CPW = NCHP // 32
NSLOT = 4
NTILES = 16
RSTEP = 3128
WD = 8

_f32 = jnp.float32


def _per_tile_rows(s, fn):
    r0 = pl.multiple_of(s * RSTEP, 8)

    @pl.when(s < NTILES - 1)
    def _():
        fn(r0, RSTEP)

    @pl.when(s == NTILES - 1)
    def _():
        fn((NTILES - 1) * RSTEP, N - (NTILES - 1) * RSTEP)


def _agg_body(h2, s2, dstp, z, agg2, agg_s,
              i0, i1, i2, i3, i4, i5, i6, i7,
              j0, j1, j2, j3, j4, j5, j6, j7,
              r0_, r1_, r2_, r3_,
              e0, e1, e2_, e3, e4, e5, e6, e7,
              g0, g1, g2, g3, s0, s1, s2_, s3):
    c = lax.axis_index("c")
    s = lax.axis_index("s")
    isrc = (i0, i1, i2, i3, i4, i5, i6, i7)
    idst = (j0, j1, j2, j3, j4, j5, j6, j7)
    rows = (r0_, r1_, r2_, r3_)
    isems = (e0, e1, e2_, e3, e4, e5, e6, e7)
    gsems = (g0, g1, g2, g3)
    ssems = (s0, s1, s2_, s3)
    _per_tile_rows(s, lambda r0, nr: pltpu.sync_copy(
        z.at[pl.ds(r0, nr)], agg_s.at[pl.ds(r0, nr)]))
    plsc.subcore_barrier()

    base = s * CPT

    def issue_idx(i, a):
        pltpu.async_copy(s2.at[c * NCHP + base + i], isrc[a], isems[a])
        pltpu.async_copy(dstp.at[base + i], idst[a], isems[a])

    def wait_idx(a):
        pltpu.make_async_copy(s2.at[0], isrc[a], isems[a]).wait()
        pltpu.make_async_copy(dstp.at[0], idst[a], isems[a]).wait()

    def issue_gather(a, b):
        pltpu.async_copy(h2.at[isrc[a]], rows[b], gsems[b])

    def wait_gather(b):
        pltpu.make_async_copy(h2.at[pl.ds(0, CH)], rows[b], gsems[b]).wait()

    def issue_scatter(a, b):
        pltpu.async_copy(rows[b], agg_s.at[idst[a]], ssems[b], add=True)

    def wait_scatter(b):
        pltpu.make_async_copy(rows[b], agg_s.at[pl.ds(0, CH)],
                              ssems[b]).wait()

    for a in range(2 * NSLOT):
        issue_idx(a, a)
    for b in range(NSLOT):
        wait_idx(b)
        issue_gather(b, b)

    def do_block(p, last):
        for h in range(2):
            for b in range(NSLOT):
                a = 4 * h + b
                i = 8 * p + a
                wait_gather(b)
                issue_scatter(a, b)
            for b in range(NSLOT):
                a = 4 * h + b
                i = 8 * p + a
                if last and h == 1:
                    wait_scatter(b)
                    continue
                wait_scatter(b)
                if not (last and h == 0):
                    issue_idx(i + 8, a)
                wait_idx(4 * (1 - h) + b)
                issue_gather(4 * (1 - h) + b, b)

    def body(p, carry):
        do_block(p, False)
        return carry

    nblk = CPT // (2 * NSLOT)
    lax.fori_loop(0, nblk - 1, body, 0)
    do_block(nblk - 1, True)

    plsc.subcore_barrier()
    _per_tile_rows(s, lambda r0, nr: pltpu.sync_copy(
        agg_s.at[pl.ds(r0, nr)],
        agg2.at[pl.ds(pl.multiple_of(c * N + r0, 8), nr)]))


def _deg_body(dstp, zd, ones, deg2, deg_s, didx, ones_v, d0, d1, d2, d3):
    c = lax.axis_index("c")
    s = lax.axis_index("s")
    dsems = (d0, d1, d2, d3)
    base = (c * 16 + s) * CPW
    pltpu.sync_copy(dstp.at[pl.ds(base, CPW)], didx)
    _per_tile_rows(s, lambda r0, nr: pltpu.sync_copy(
        zd.at[pl.ds(r0, nr)], deg_s.at[pl.ds(r0, nr)]))
    pltpu.sync_copy(ones, ones_v)
    plsc.subcore_barrier()

    def issue(j, b):
        pltpu.async_copy(ones_v, deg_s.at[didx.at[j]], dsems[b], add=True)

    def wait(b):
        pltpu.make_async_copy(ones_v, deg_s.at[pl.ds(0, CH)],
                              dsems[b]).wait()

    for b in range(NSLOT):
        issue(b, b)

    def body(q, carry):
        for b in range(NSLOT):
            wait(b)
            issue(q * NSLOT + b, b)
        return carry

    lax.fori_loop(1, CPW // NSLOT, body, 0)
    for b in range(NSLOT):
        wait(b)
    plsc.subcore_barrier()
    _per_tile_rows(s, lambda r0, nr: pltpu.sync_copy(
        deg_s.at[pl.ds(r0, nr)],
        deg2.at[pl.ds(pl.multiple_of(c * N + r0, 8), nr)]))


def _edge_gather_body(pq2, e2, s_out,
                      sidx, didx, a0, a1, a2, a3, b0, b1, b2, b3,
                      ga0, ga1, ga2, ga3, gb0, gb1, gb2, gb3,
                      t0, t1, t2, t3):
    c = lax.axis_index("c")
    s = lax.axis_index("s")
    w = s * 2 + c
    rows_a = (a0, a1, a2, a3)
    rows_b = (b0, b1, b2, b3)
    gsa = (ga0, ga1, ga2, ga3)
    gsb = (gb0, gb1, gb2, gb3)
    sts = (t0, t1, t2, t3)
    pltpu.sync_copy(e2.at[pl.ds(w * CPW, CPW)], sidx)
    pltpu.sync_copy(e2.at[pl.ds(NCHP + w * CPW, CPW)], didx)
    base = w * CPW

    def issue_a(j, b):
        pltpu.async_copy(pq2.at[sidx.at[j]], rows_a[b], gsa[b])

    def issue_b(j, b):
        pltpu.async_copy(pq2.at[didx.at[j]], rows_b[b], gsb[b])

    nq = CPW // NSLOT

    def process_quad(q, reissue):
        for b in range(NSLOT):
            j = q * NSLOT + b
            g = base + j
            pltpu.make_async_copy(pq2.at[pl.ds(0, CH)], rows_a[b],
                                  gsa[b]).wait()
            pltpu.make_async_copy(pq2.at[pl.ds(0, CH)], rows_b[b],
                                  gsb[b]).wait()

            def add_row(r, carry, _b=b):
                for k in range(H // 16):
                    sl = pl.ds(k * 16, 16)
                    rows_b[_b][r, sl] = rows_b[_b][r, sl] + rows_a[_b][r, sl]
                return carry

            lax.fori_loop(0, CH, add_row, 0)
            if reissue:
                issue_a(j + NSLOT, b)

            @pl.when(g < NCHUNK)
            def _():
                pltpu.async_copy(
                    rows_b[b],
                    s_out.at[pl.ds(pl.multiple_of(g * CH, 8), CH)], sts[b])
        if reissue:
            for b in range(NSLOT):
                j = q * NSLOT + b
                g = base + j

                @pl.when(g < NCHUNK)
                def _():
                    pltpu.make_async_copy(rows_b[b], s_out.at[pl.ds(0, CH)],
                                          sts[b]).wait()
                issue_b(j + NSLOT, b)

    for b in range(NSLOT):
        issue_a(b, b)
        issue_b(b, b)

    def body(q, carry):
        process_quad(q, True)
        return carry

    lax.fori_loop(0, nq - 1, body, 0)
    process_quad(nq - 1, False)
    for b in range(NSLOT):
        g = base + (nq - 1) * NSLOT + b

        @pl.when(g < NCHUNK)
        def _():
            pltpu.make_async_copy(rows_b[b], s_out.at[pl.ds(0, CH)],
                                  sts[b]).wait()


@functools.cache
def _sc_calls():
    mesh = plsc.VectorSubcoreMesh(core_axis_name="c", subcore_axis_name="s")
    params = pltpu.CompilerParams(use_tc_tiling_on_sc=False)
    idx_scratch = [pltpu.VMEM((CH,), jnp.int32)] * (2 * NSLOT)
    rows_scratch = [pltpu.VMEM((CH, HH), _f32)] * NSLOT
    sems = [pltpu.SemaphoreType.DMA] * NSLOT
    agg = pl.kernel(
        _agg_body,
        compiler_params=params,
        out_type=jax.ShapeDtypeStruct((2 * N, HH), _f32),
        mesh=mesh,
        scratch_types=[
            pltpu.VMEM_SHARED((N + 8, HH), _f32),
            *idx_scratch, *idx_scratch,
            *rows_scratch,
            *sems, *sems, *sems, *sems,
        ],
    )
    deg = pl.kernel(
        _deg_body,
        compiler_params=params,
        out_type=jax.ShapeDtypeStruct((2 * N, WD), _f32),
        mesh=mesh,
        scratch_types=[
            pltpu.VMEM_SHARED((N + 8, WD), _f32),
            pltpu.VMEM((CPW, CH), jnp.int32),
            pltpu.VMEM((CH, WD), _f32),
            *sems,
        ],
    )
    edge_gather = pl.kernel(
        _edge_gather_body,
        compiler_params=params,
        out_type=jax.ShapeDtypeStruct((E, H), _f32),
        mesh=mesh,
        scratch_types=[
            pltpu.VMEM((CPW, CH), jnp.int32),
            pltpu.VMEM((CPW, CH), jnp.int32),
            *([pltpu.VMEM((CH, H), _f32)] * (2 * NSLOT)),
            *sems, *sems, *sems,
        ],
    )
    return agg, deg, edge_gather


_BN = 1000
_BE = 2000


def _proj_body(x_ref, wpt_ref, bp_ref, out_ref):
    h = jnp.dot(x_ref[...], wpt_ref[...], preferred_element_type=_f32)
    h = h + bp_ref[...]
    out_ref[0] = h[:, :HH]
    out_ref[1] = h[:, HH:]


def _sage_update(agg_a, agg_b, deg_a, deg_b, h_a, h_b, wlt, bl, wrt, g, be):
    agg = jnp.concatenate([agg_a[...], agg_b[...]], axis=1)
    d = jnp.maximum(deg_a[...][:, :1] + deg_b[...][:, :1], 1.0)
    mean = agg / d
    h = jnp.concatenate([h_a[...], h_b[...]], axis=1)
    t = (jnp.dot(mean, wlt[...], preferred_element_type=_f32) + bl[...]
         + jnp.dot(h, wrt[...], preferred_element_type=_f32))
    mu = jnp.mean(t, axis=1, keepdims=True)
    var = jnp.mean((t - mu) * (t - mu), axis=1, keepdims=True)
    y = (t - mu) * lax.rsqrt(var + 1e-5) * g[...] + be[...]
    return jnp.maximum(y, 0.0)


def _update_body(agg_a, agg_b, deg_a, deg_b, h_a, h_b, wlt, bl, wrt, g, be,
                 out_ref):
    y = _sage_update(agg_a, agg_b, deg_a, deg_b, h_a, h_b, wlt, bl, wrt, g,
                     be)
    out_ref[0] = y[:, :HH]
    out_ref[1] = y[:, HH:]


def _update_pq_body(agg_a, agg_b, deg_a, deg_b, h_a, h_b, wlt, bl, wrt, g, be,
                    w1at, w1bt, b1, out_ref):
    y = _sage_update(agg_a, agg_b, deg_a, deg_b, h_a, h_b, wlt, bl, wrt, g,
                     be)
    out_ref[0] = jnp.dot(y, w1at[...], preferred_element_type=_f32)
    out_ref[1] = jnp.dot(y, w1bt[...], preferred_element_type=_f32) + b1[...]


def _softplus(x):
    return jnp.maximum(x, 0.0) + jnp.log(1.0 + jnp.exp(-jnp.abs(x)))


def _edge_mlp_body(s2_ref, ea2_ref, w1ct2, w2t2, b2_2, w3r2, b3, out_ref):
    z1 = jnp.maximum(
        s2_ref[...] + jnp.dot(ea2_ref[...], w1ct2[...],
                              preferred_element_type=_f32), 0.0)
    z2 = jnp.maximum(jnp.dot(z1, w2t2[...], preferred_element_type=_f32)
                     + b2_2[...], 0.0)
    t = z2 * w3r2[...]
    out_ref[0] = _softplus(jnp.sum(t[:, :HH], axis=1) + b3[0, 0])
    out_ref[1] = _softplus(jnp.sum(t[:, HH:], axis=1) + b3[0, 0])


def _halved(i):
    return (i, 0)


def _halved_hi(i):
    return (N // _BN + i, 0)


def _full(i):
    return (0, 0)


_h2_spec = [pl.BlockSpec((_BN, HH), _halved), pl.BlockSpec((_BN, HH), _halved_hi)]
_w64_spec = pl.BlockSpec((H, H), _full)
_row64_spec = pl.BlockSpec((1, H), _full)


def _proj_call(x, wpt, bp2):
    out = pl.pallas_call(
        _proj_body,
        grid=(N // _BN,),
        in_specs=[pl.BlockSpec((_BN, 128), _halved), pl.BlockSpec((128, H), _full),
                  _row64_spec],
        out_specs=pl.BlockSpec((2, _BN, HH), lambda i: (0, i, 0)),
        out_shape=jax.ShapeDtypeStruct((2, N, HH), _f32),
    )(x, wpt, bp2)
    return out.reshape(2 * N, HH)


_deg_spec = [pl.BlockSpec((_BN, WD), _halved), pl.BlockSpec((_BN, WD), _halved_hi)]


def _update_call(agg2, deg2, h2, wlt, bl2, wrt, g2, be2):
    out = pl.pallas_call(
        _update_body,
        grid=(N // _BN,),
        in_specs=_h2_spec + _deg_spec + _h2_spec
        + [_w64_spec, _row64_spec, _w64_spec, _row64_spec, _row64_spec],
        out_specs=pl.BlockSpec((2, _BN, HH), lambda i: (0, i, 0)),
        out_shape=jax.ShapeDtypeStruct((2, N, HH), _f32),
    )(agg2, agg2, deg2, deg2, h2, h2, wlt, bl2, wrt, g2, be2)
    return out.reshape(2 * N, HH)


def _update_pq_call(agg2, deg2, h2, wlt, bl2, wrt, g2, be2, w1at, w1bt, b12):
    out = pl.pallas_call(
        _update_pq_body,
        grid=(N // _BN,),
        in_specs=_h2_spec + _deg_spec + _h2_spec
        + [_w64_spec, _row64_spec, _w64_spec, _row64_spec, _row64_spec,
           _w64_spec, _w64_spec, _row64_spec],
        out_specs=pl.BlockSpec((2, _BN, H), lambda i: (0, i, 0)),
        out_shape=jax.ShapeDtypeStruct((2, N, H), _f32),
    )(agg2, agg2, deg2, deg2, h2, h2, wlt, bl2, wrt, g2, be2, w1at, w1bt, b12)
    return out.reshape(2 * N, H)


_BE2 = 3200


def _edge_mlp_call(s_arr, edge_attr, w1ct, w2t, b2, w3row, b32):
    s2d = s_arr.reshape(E // 2, 2 * H)
    ea2 = edge_attr.reshape(E // 2, 32)
    w1ct2 = jnp.zeros((32, 2 * H), _f32)
    w1ct2 = w1ct2.at[:16, :H].set(w1ct).at[16:, H:].set(w1ct)
    w2t2 = jnp.zeros((2 * H, H), _f32)
    w2t2 = w2t2.at[:H, :HH].set(w2t).at[H:, HH:].set(w2t)
    b2_2 = jnp.concatenate([b2, b2]).reshape(1, H)
    w3r2 = jnp.concatenate([w3row, w3row]).reshape(1, H)
    out = pl.pallas_call(
        _edge_mlp_body,
        grid=(E // 2 // _BE2,),
        in_specs=[pl.BlockSpec((_BE2, 2 * H), _halved),
                  pl.BlockSpec((_BE2, 32), _halved),
                  pl.BlockSpec((32, 2 * H), _full),
                  pl.BlockSpec((2 * H, H), _full),
                  pl.BlockSpec((1, H), _full),
                  pl.BlockSpec((1, H), _full),
                  pl.BlockSpec((1, 1), _full)],
        out_specs=pl.BlockSpec((2, _BE2), lambda i: (0, i)),
        out_shape=jax.ShapeDtypeStruct((2, E // 2), _f32),
    )(s2d, ea2, w1ct2, w2t2, b2_2, w3r2, b32)
    return jnp.transpose(out).reshape(E)


def kernel(x, edge_index, edge_attr, Wp, bp,
           Wl0, bl0, Wr0, g0, be0,
           Wl1, bl1, Wr1, g1, be1,
           Wl2, bl2, Wr2, g2, be2,
           W1, b1, W2, b2, W3, b3):
    src = edge_index[0]
    dst = edge_index[1]
    pad0 = jnp.zeros((EP - E,), jnp.int32)
    padn = jnp.full((EP - E,), N, jnp.int32)
    src_a = jnp.concatenate([src, pad0]).reshape(NCHP, CH)
    src_b = jnp.concatenate([src + N, padn]).reshape(NCHP, CH)
    dstp = jnp.concatenate([dst, padn]).reshape(NCHP, CH)
    dstn = jnp.concatenate([dst + N, padn]).reshape(NCHP, CH)
    s2 = jnp.concatenate([src_a, src_b], 0)
    e2 = jnp.concatenate([src_a, dstn], 0)
    z = jnp.zeros((N, HH), _f32)
    zd = jnp.zeros((N, WD), _f32)
    ones = jnp.ones((CH, WD), _f32)

    agg_call, deg_call, edge_gather_call = _sc_calls()

    h2 = _proj_call(x, Wp.T, bp.reshape(1, H))
    deg2 = deg_call(dstp, zd, ones)

    agg2 = agg_call(h2, s2, dstp, z)
    h2 = _update_call(agg2, deg2, h2, Wl0.T, bl0.reshape(1, H), Wr0.T,
                      g0.reshape(1, H), be0.reshape(1, H))

    agg2 = agg_call(h2, s2, dstp, z)
    h2 = _update_call(agg2, deg2, h2, Wl1.T, bl1.reshape(1, H), Wr1.T,
                      g1.reshape(1, H), be1.reshape(1, H))

    agg2 = agg_call(h2, s2, dstp, z)
    w1at = W1[:, :H].T
    w1bt = W1[:, H:2 * H].T
    pq2 = _update_pq_call(agg2, deg2, h2, Wl2.T, bl2.reshape(1, H), Wr2.T,
                          g2.reshape(1, H), be2.reshape(1, H),
                          w1at, w1bt, b1.reshape(1, H))

    s_arr = edge_gather_call(pq2, e2)

    return _edge_mlp_call(s_arr, edge_attr, W1[:, 2 * H:].T, W2.T,
                          b2, W3[0], b3.reshape(1, 1))


# speedup vs baseline: 1.2284x; 1.2284x over previous
import functools

import jax
import jax.numpy as jnp
from jax import lax
from jax.experimental import pallas as pl
from jax.experimental.pallas import tpu as pltpu
from jax.experimental.pallas import tpu_sc as plsc

N = 50000
E = 800000
H = 64
HH = 32
CH = 128
NCHUNK = E // CH
NCHP = 6272
EP = NCHP * CH
CPT = NCHP // 16
CPW = NCHP // 32
NSLOT = 4
NTILES = 16
RSTEP = 3128
WD = 8

_f32 = jnp.float32


def _per_tile_rows(s, fn):
    r0 = pl.multiple_of(s * RSTEP, 8)

    @pl.when(s < NTILES - 1)
    def _():
        fn(r0, RSTEP)

    @pl.when(s == NTILES - 1)
    def _():
        fn((NTILES - 1) * RSTEP, N - (NTILES - 1) * RSTEP)


def _agg_body(h2, s2, dstp, z, agg2, agg_s,
              i0, i1, i2, i3, i4, i5, i6, i7,
              j0, j1, j2, j3, j4, j5, j6, j7,
              r0_, r1_, r2_, r3_,
              e0, e1, e2_, e3, e4, e5, e6, e7,
              g0, g1, g2, g3, s0, s1, s2_, s3):
    c = lax.axis_index("c")
    s = lax.axis_index("s")
    isrc = (i0, i1, i2, i3, i4, i5, i6, i7)
    idst = (j0, j1, j2, j3, j4, j5, j6, j7)
    rows = (r0_, r1_, r2_, r3_)
    isems = (e0, e1, e2_, e3, e4, e5, e6, e7)
    gsems = (g0, g1, g2, g3)
    ssems = (s0, s1, s2_, s3)
    _per_tile_rows(s, lambda r0, nr: pltpu.sync_copy(
        z.at[pl.ds(r0, nr)], agg_s.at[pl.ds(r0, nr)]))
    plsc.subcore_barrier()

    base = s * CPT

    def issue_idx(i, a):
        pltpu.async_copy(s2.at[c * NCHP + base + i], isrc[a], isems[a])
        pltpu.async_copy(dstp.at[base + i], idst[a], isems[a])

    def wait_idx(a):
        pltpu.make_async_copy(s2.at[0], isrc[a], isems[a]).wait()
        pltpu.make_async_copy(dstp.at[0], idst[a], isems[a]).wait()

    def issue_gather(a, b):
        pltpu.async_copy(h2.at[isrc[a]], rows[b], gsems[b])

    def wait_gather(b):
        pltpu.make_async_copy(h2.at[pl.ds(0, CH)], rows[b], gsems[b]).wait()

    def issue_scatter(a, b):
        pltpu.async_copy(rows[b], agg_s.at[idst[a]], ssems[b], add=True)

    def wait_scatter(b):
        pltpu.make_async_copy(rows[b], agg_s.at[pl.ds(0, CH)],
                              ssems[b]).wait()

    for a in range(2 * NSLOT):
        issue_idx(a, a)
    for b in range(NSLOT):
        wait_idx(b)
        issue_gather(b, b)

    def do_block(p, last):
        for h in range(2):
            for b in range(NSLOT):
                a = 4 * h + b
                i = 8 * p + a
                wait_gather(b)
                issue_scatter(a, b)
            for b in range(NSLOT):
                a = 4 * h + b
                i = 8 * p + a
                if last and h == 1:
                    wait_scatter(b)
                    continue
                wait_scatter(b)
                if not (last and h == 0):
                    issue_idx(i + 8, a)
                wait_idx(4 * (1 - h) + b)
                issue_gather(4 * (1 - h) + b, b)

    def body(p, carry):
        do_block(p, False)
        return carry

    nblk = CPT // (2 * NSLOT)
    lax.fori_loop(0, nblk - 1, body, 0)
    do_block(nblk - 1, True)

    plsc.subcore_barrier()
    _per_tile_rows(s, lambda r0, nr: pltpu.sync_copy(
        agg_s.at[pl.ds(r0, nr)],
        agg2.at[pl.ds(pl.multiple_of(c * N + r0, 8), nr)]))


def _deg_body(dstp, zd, ones, deg2, deg_s, didx, ones_v, d0, d1, d2, d3):
    c = lax.axis_index("c")
    s = lax.axis_index("s")
    dsems = (d0, d1, d2, d3)
    base = (c * 16 + s) * CPW
    pltpu.sync_copy(dstp.at[pl.ds(base, CPW)], didx)
    _per_tile_rows(s, lambda r0, nr: pltpu.sync_copy(
        zd.at[pl.ds(r0, nr)], deg_s.at[pl.ds(r0, nr)]))
    pltpu.sync_copy(ones, ones_v)
    plsc.subcore_barrier()

    def issue(j, b):
        pltpu.async_copy(ones_v, deg_s.at[didx.at[j]], dsems[b], add=True)

    def wait(b):
        pltpu.make_async_copy(ones_v, deg_s.at[pl.ds(0, CH)],
                              dsems[b]).wait()

    for b in range(NSLOT):
        issue(b, b)

    def body(q, carry):
        for b in range(NSLOT):
            wait(b)
            issue(q * NSLOT + b, b)
        return carry

    lax.fori_loop(1, CPW // NSLOT, body, 0)
    for b in range(NSLOT):
        wait(b)
    plsc.subcore_barrier()
    _per_tile_rows(s, lambda r0, nr: pltpu.sync_copy(
        deg_s.at[pl.ds(r0, nr)],
        deg2.at[pl.ds(pl.multiple_of(c * N + r0, 8), nr)]))


def _edge_gather_body(pq2, e2, s_out,
                      sidx, didx, a0, a1, a2, a3, b0, b1, b2, b3,
                      ga0, ga1, ga2, ga3, gb0, gb1, gb2, gb3,
                      t0, t1, t2, t3):
    c = lax.axis_index("c")
    s = lax.axis_index("s")
    w = s * 2 + c
    rows_a = (a0, a1, a2, a3)
    rows_b = (b0, b1, b2, b3)
    gsa = (ga0, ga1, ga2, ga3)
    gsb = (gb0, gb1, gb2, gb3)
    sts = (t0, t1, t2, t3)
    pltpu.sync_copy(e2.at[pl.ds(w * CPW, CPW)], sidx)
    pltpu.sync_copy(e2.at[pl.ds(NCHP + w * CPW, CPW)], didx)
    base = w * CPW

    def issue_a(j, b):
        pltpu.async_copy(pq2.at[sidx.at[j]], rows_a[b], gsa[b])

    def issue_b(j, b):
        pltpu.async_copy(pq2.at[didx.at[j]], rows_b[b], gsb[b])

    nq = CPW // NSLOT

    def process_quad(q, reissue):
        for b in range(NSLOT):
            j = q * NSLOT + b
            g = base + j
            pltpu.make_async_copy(pq2.at[pl.ds(0, CH)], rows_a[b],
                                  gsa[b]).wait()
            pltpu.make_async_copy(pq2.at[pl.ds(0, CH)], rows_b[b],
                                  gsb[b]).wait()

            def add_row(r, carry, _b=b):
                for k in range(H // 16):
                    sl = pl.ds(k * 16, 16)
                    rows_b[_b][r, sl] = rows_b[_b][r, sl] + rows_a[_b][r, sl]
                return carry

            lax.fori_loop(0, CH, add_row, 0)
            if reissue:
                issue_a(j + NSLOT, b)

            @pl.when(g < NCHUNK)
            def _():
                pltpu.async_copy(
                    rows_b[b],
                    s_out.at[pl.ds(pl.multiple_of(g * CH, 8), CH)], sts[b])
        if reissue:
            for b in range(NSLOT):
                j = q * NSLOT + b
                g = base + j

                @pl.when(g < NCHUNK)
                def _():
                    pltpu.make_async_copy(rows_b[b], s_out.at[pl.ds(0, CH)],
                                          sts[b]).wait()
                issue_b(j + NSLOT, b)

    for b in range(NSLOT):
        issue_a(b, b)
        issue_b(b, b)

    def body(q, carry):
        process_quad(q, True)
        return carry

    lax.fori_loop(0, nq - 1, body, 0)
    process_quad(nq - 1, False)
    for b in range(NSLOT):
        g = base + (nq - 1) * NSLOT + b

        @pl.when(g < NCHUNK)
        def _():
            pltpu.make_async_copy(rows_b[b], s_out.at[pl.ds(0, CH)],
                                  sts[b]).wait()


@functools.cache
def _sc_calls():
    mesh = plsc.VectorSubcoreMesh(core_axis_name="c", subcore_axis_name="s")
    params = pltpu.CompilerParams(use_tc_tiling_on_sc=False)
    idx_scratch = [pltpu.VMEM((CH,), jnp.int32)] * (2 * NSLOT)
    rows_scratch = [pltpu.VMEM((CH, HH), _f32)] * NSLOT
    sems = [pltpu.SemaphoreType.DMA] * NSLOT
    agg = pl.kernel(
        _agg_body,
        compiler_params=params,
        out_type=jax.ShapeDtypeStruct((2 * N, HH), _f32),
        mesh=mesh,
        scratch_types=[
            pltpu.VMEM_SHARED((N + 8, HH), _f32),
            *idx_scratch, *idx_scratch,
            *rows_scratch,
            *sems, *sems, *sems, *sems,
        ],
    )
    deg = pl.kernel(
        _deg_body,
        compiler_params=params,
        out_type=jax.ShapeDtypeStruct((2 * N, WD), _f32),
        mesh=mesh,
        scratch_types=[
            pltpu.VMEM_SHARED((N + 8, WD), _f32),
            pltpu.VMEM((CPW, CH), jnp.int32),
            pltpu.VMEM((CH, WD), _f32),
            *sems,
        ],
    )
    edge_gather = pl.kernel(
        _edge_gather_body,
        compiler_params=params,
        out_type=jax.ShapeDtypeStruct((E, H), _f32),
        mesh=mesh,
        scratch_types=[
            pltpu.VMEM((CPW, CH), jnp.int32),
            pltpu.VMEM((CPW, CH), jnp.int32),
            *([pltpu.VMEM((CH, H), _f32)] * (2 * NSLOT)),
            *sems, *sems, *sems,
        ],
    )
    return agg, deg, edge_gather


_BN = 1000
_BE = 2000


def _proj_body(x_ref, wpt_ref, bp_ref, out_ref):
    h = jnp.dot(x_ref[...], wpt_ref[...], preferred_element_type=_f32)
    h = h + bp_ref[...]
    out_ref[0] = h[:, :HH]
    out_ref[1] = h[:, HH:]


def _sage_update(agg_a, agg_b, deg_a, deg_b, h_a, h_b, wlt, bl, wrt, g, be):
    agg = jnp.concatenate([agg_a[...], agg_b[...]], axis=1)
    d = jnp.maximum(deg_a[...][:, :1] + deg_b[...][:, :1], 1.0)
    mean = agg / d
    h = jnp.concatenate([h_a[...], h_b[...]], axis=1)
    t = (jnp.dot(mean, wlt[...], preferred_element_type=_f32) + bl[...]
         + jnp.dot(h, wrt[...], preferred_element_type=_f32))
    mu = jnp.mean(t, axis=1, keepdims=True)
    var = jnp.mean((t - mu) * (t - mu), axis=1, keepdims=True)
    y = (t - mu) * lax.rsqrt(var + 1e-5) * g[...] + be[...]
    return jnp.maximum(y, 0.0)


def _update_body(agg_a, agg_b, deg_a, deg_b, h_a, h_b, wlt, bl, wrt, g, be,
                 out_ref):
    y = _sage_update(agg_a, agg_b, deg_a, deg_b, h_a, h_b, wlt, bl, wrt, g,
                     be)
    out_ref[0] = y[:, :HH]
    out_ref[1] = y[:, HH:]


def _update_pq_body(agg_a, agg_b, deg_a, deg_b, h_a, h_b, wlt, bl, wrt, g, be,
                    w1at, w1bt, b1, out_ref):
    y = _sage_update(agg_a, agg_b, deg_a, deg_b, h_a, h_b, wlt, bl, wrt, g,
                     be)
    out_ref[0] = jnp.dot(y, w1at[...], preferred_element_type=_f32)
    out_ref[1] = jnp.dot(y, w1bt[...], preferred_element_type=_f32) + b1[...]


def _softplus(x):
    return jnp.maximum(x, 0.0) + jnp.log(1.0 + jnp.exp(-jnp.abs(x)))


def _edge_mlp_body(s2_ref, ea_lo, ea_hi, w1ct, w2t2, b2_2, w3t2, b3, out_ref):
    lo = jnp.dot(ea_lo[...], w1ct[...], preferred_element_type=_f32)
    hi = jnp.dot(ea_hi[...], w1ct[...], preferred_element_type=_f32)
    z1 = jnp.maximum(s2_ref[...] + jnp.concatenate([lo, hi], axis=1), 0.0)
    z2 = jnp.maximum(jnp.dot(z1, w2t2[...], preferred_element_type=_f32)
                     + b2_2[...], 0.0)
    p = jnp.dot(z2, w3t2[...], preferred_element_type=_f32) + b3[0, 0]
    out_ref[...] = _softplus(p.T)


def _halved(i):
    return (i, 0)


def _halved_hi(i):
    return (N // _BN + i, 0)


def _full(i):
    return (0, 0)


_h2_spec = [pl.BlockSpec((_BN, HH), _halved), pl.BlockSpec((_BN, HH), _halved_hi)]
_w64_spec = pl.BlockSpec((H, H), _full)
_row64_spec = pl.BlockSpec((1, H), _full)


def _proj_call(x, wpt, bp2):
    out = pl.pallas_call(
        _proj_body,
        grid=(N // _BN,),
        in_specs=[pl.BlockSpec((_BN, 128), _halved), pl.BlockSpec((128, H), _full),
                  _row64_spec],
        out_specs=pl.BlockSpec((2, _BN, HH), lambda i: (0, i, 0)),
        out_shape=jax.ShapeDtypeStruct((2, N, HH), _f32),
    )(x, wpt, bp2)
    return out.reshape(2 * N, HH)


_deg_spec = [pl.BlockSpec((_BN, WD), _halved), pl.BlockSpec((_BN, WD), _halved_hi)]


def _update_call(agg2, deg2, h2, wlt, bl2, wrt, g2, be2):
    out = pl.pallas_call(
        _update_body,
        grid=(N // _BN,),
        in_specs=_h2_spec + _deg_spec + _h2_spec
        + [_w64_spec, _row64_spec, _w64_spec, _row64_spec, _row64_spec],
        out_specs=pl.BlockSpec((2, _BN, HH), lambda i: (0, i, 0)),
        out_shape=jax.ShapeDtypeStruct((2, N, HH), _f32),
    )(agg2, agg2, deg2, deg2, h2, h2, wlt, bl2, wrt, g2, be2)
    return out.reshape(2 * N, HH)


def _update_pq_call(agg2, deg2, h2, wlt, bl2, wrt, g2, be2, w1at, w1bt, b12):
    out = pl.pallas_call(
        _update_pq_body,
        grid=(N // _BN,),
        in_specs=_h2_spec + _deg_spec + _h2_spec
        + [_w64_spec, _row64_spec, _w64_spec, _row64_spec, _row64_spec,
           _w64_spec, _w64_spec, _row64_spec],
        out_specs=pl.BlockSpec((2, _BN, H), lambda i: (0, i, 0)),
        out_shape=jax.ShapeDtypeStruct((2, N, H), _f32),
    )(agg2, agg2, deg2, deg2, h2, h2, wlt, bl2, wrt, g2, be2, w1at, w1bt, b12)
    return out.reshape(2 * N, H)


_BE2 = 3200


def _edge_mlp_call(s_arr, edge_attr, w1ct, w2t, b2, w3row, b32):
    s2d = s_arr.reshape(E // 2, 2 * H)
    w2t2 = jnp.zeros((2 * H, H), _f32)
    w2t2 = w2t2.at[:H, :HH].set(w2t).at[H:, HH:].set(w2t)
    b2_2 = jnp.concatenate([b2, b2]).reshape(1, H)
    w3t2 = jnp.zeros((H, 2), _f32)
    w3t2 = w3t2.at[:HH, 0].set(w3row).at[HH:, 1].set(w3row)
    ngrid = E // 2 // _BE2
    out = pl.pallas_call(
        _edge_mlp_body,
        grid=(ngrid,),
        in_specs=[pl.BlockSpec((_BE2, 2 * H), _halved),
                  pl.BlockSpec((_BE2, 16), _halved),
                  pl.BlockSpec((_BE2, 16), lambda i: (ngrid + i, 0)),
                  pl.BlockSpec((16, H), _full),
                  pl.BlockSpec((2 * H, H), _full),
                  pl.BlockSpec((1, H), _full),
                  pl.BlockSpec((H, 2), _full),
                  pl.BlockSpec((1, 1), _full)],
        out_specs=pl.BlockSpec((2, _BE2), lambda i: (0, i)),
        out_shape=jax.ShapeDtypeStruct((2, E // 2), _f32),
    )(s2d, edge_attr, edge_attr, w1ct, w2t2, b2_2, w3t2, b32)
    return out.reshape(E)


def kernel(x, edge_index, edge_attr, Wp, bp,
           Wl0, bl0, Wr0, g0, be0,
           Wl1, bl1, Wr1, g1, be1,
           Wl2, bl2, Wr2, g2, be2,
           W1, b1, W2, b2, W3, b3):
    src = edge_index[0]
    dst = edge_index[1]
    pad0 = jnp.zeros((EP - E,), jnp.int32)
    padn = jnp.full((EP - E,), N, jnp.int32)
    src_a = jnp.concatenate([src, pad0]).reshape(NCHP, CH)
    src_b = jnp.concatenate([src + N, padn]).reshape(NCHP, CH)
    dstp = jnp.concatenate([dst, padn]).reshape(NCHP, CH)
    s2 = jnp.concatenate([src_a, src_b], 0)
    dn = dst + N
    src_p = jnp.stack([src[:E // 2], src[E // 2:]], 1).reshape(E)
    dn_p = jnp.stack([dn[:E // 2], dn[E // 2:]], 1).reshape(E)
    e2 = jnp.concatenate([
        jnp.concatenate([src_p, pad0]).reshape(NCHP, CH),
        jnp.concatenate([dn_p, padn]).reshape(NCHP, CH)], 0)
    z = jnp.zeros((N, HH), _f32)
    zd = jnp.zeros((N, WD), _f32)
    ones = jnp.ones((CH, WD), _f32)

    agg_call, deg_call, edge_gather_call = _sc_calls()

    h2 = _proj_call(x, Wp.T, bp.reshape(1, H))
    deg2 = deg_call(dstp, zd, ones)

    agg2 = agg_call(h2, s2, dstp, z)
    h2 = _update_call(agg2, deg2, h2, Wl0.T, bl0.reshape(1, H), Wr0.T,
                      g0.reshape(1, H), be0.reshape(1, H))

    agg2 = agg_call(h2, s2, dstp, z)
    h2 = _update_call(agg2, deg2, h2, Wl1.T, bl1.reshape(1, H), Wr1.T,
                      g1.reshape(1, H), be1.reshape(1, H))

    agg2 = agg_call(h2, s2, dstp, z)
    w1at = W1[:, :H].T
    w1bt = W1[:, H:2 * H].T
    pq2 = _update_pq_call(agg2, deg2, h2, Wl2.T, bl2.reshape(1, H), Wr2.T,
                          g2.reshape(1, H), be2.reshape(1, H),
                          w1at, w1bt, b1.reshape(1, H))

    s_arr = edge_gather_call(pq2, e2)

    return _edge_mlp_call(s_arr, edge_attr, W1[:, 2 * H:].T, W2.T,
                          b2, W3[0], b3.reshape(1, 1))


# speedup vs baseline: 1.3713x; 1.1164x over previous
import functools

import jax
import jax.numpy as jnp
from jax import lax
from jax.experimental import pallas as pl
from jax.experimental.pallas import tpu as pltpu
from jax.experimental.pallas import tpu_sc as plsc

N = 50000
E = 800000
H = 64
HH = 32
CH = 128
NCHUNK = E // CH
NCHP = 6272
EP = NCHP * CH
CPT = NCHP // 16
CPW = NCHP // 32
NSLOT = 4
NTILES = 16
RSTEP = 3128
WD = 8

_f32 = jnp.float32


def _per_tile_rows(s, fn):
    r0 = pl.multiple_of(s * RSTEP, 8)

    @pl.when(s < NTILES - 1)
    def _():
        fn(r0, RSTEP)

    @pl.when(s == NTILES - 1)
    def _():
        fn((NTILES - 1) * RSTEP, N - (NTILES - 1) * RSTEP)


def _agg_body(h2, s2, dstp, z, agg2, agg_s,
              i0, i1, i2, i3, i4, i5, i6, i7,
              j0, j1, j2, j3, j4, j5, j6, j7,
              r0_, r1_, r2_, r3_,
              e0, e1, e2_, e3, e4, e5, e6, e7,
              g0, g1, g2, g3, s0, s1, s2_, s3):
    c = lax.axis_index("c")
    s = lax.axis_index("s")
    isrc = (i0, i1, i2, i3, i4, i5, i6, i7)
    idst = (j0, j1, j2, j3, j4, j5, j6, j7)
    rows = (r0_, r1_, r2_, r3_)
    isems = (e0, e1, e2_, e3, e4, e5, e6, e7)
    gsems = (g0, g1, g2, g3)
    ssems = (s0, s1, s2_, s3)
    _per_tile_rows(s, lambda r0, nr: pltpu.sync_copy(
        z.at[pl.ds(r0, nr)], agg_s.at[pl.ds(r0, nr)]))
    plsc.subcore_barrier()

    base = s * CPT

    def issue_idx(i, a):
        pltpu.async_copy(s2.at[c * NCHP + base + i], isrc[a], isems[a])
        pltpu.async_copy(dstp.at[base + i], idst[a], isems[a])

    def wait_idx(a):
        pltpu.make_async_copy(s2.at[0], isrc[a], isems[a]).wait()
        pltpu.make_async_copy(dstp.at[0], idst[a], isems[a]).wait()

    def issue_gather(a, b):
        pltpu.async_copy(h2.at[isrc[a]], rows[b], gsems[b])

    def wait_gather(b):
        pltpu.make_async_copy(h2.at[pl.ds(0, CH)], rows[b], gsems[b]).wait()

    def issue_scatter(a, b):
        pltpu.async_copy(rows[b], agg_s.at[idst[a]], ssems[b], add=True)

    def wait_scatter(b):
        pltpu.make_async_copy(rows[b], agg_s.at[pl.ds(0, CH)],
                              ssems[b]).wait()

    for a in range(2 * NSLOT):
        issue_idx(a, a)
    for b in range(NSLOT):
        wait_idx(b)
        issue_gather(b, b)

    def do_block(p, last):
        for h in range(2):
            for b in range(NSLOT):
                a = 4 * h + b
                i = 8 * p + a
                wait_gather(b)
                issue_scatter(a, b)
            for b in range(NSLOT):
                a = 4 * h + b
                i = 8 * p + a
                if last and h == 1:
                    wait_scatter(b)
                    continue
                wait_scatter(b)
                if not (last and h == 0):
                    issue_idx(i + 8, a)
                wait_idx(4 * (1 - h) + b)
                issue_gather(4 * (1 - h) + b, b)

    def body(p, carry):
        do_block(p, False)
        return carry

    nblk = CPT // (2 * NSLOT)
    lax.fori_loop(0, nblk - 1, body, 0)
    do_block(nblk - 1, True)

    plsc.subcore_barrier()
    _per_tile_rows(s, lambda r0, nr: pltpu.sync_copy(
        agg_s.at[pl.ds(r0, nr)],
        agg2.at[pl.ds(pl.multiple_of(c * N + r0, 8), nr)]))


def _deg_body(dstp, zd, ones, deg2, deg_s, didx, ones_v, d0, d1, d2, d3):
    c = lax.axis_index("c")
    s = lax.axis_index("s")
    dsems = (d0, d1, d2, d3)
    base = (c * 16 + s) * CPW
    pltpu.sync_copy(dstp.at[pl.ds(base, CPW)], didx)
    _per_tile_rows(s, lambda r0, nr: pltpu.sync_copy(
        zd.at[pl.ds(r0, nr)], deg_s.at[pl.ds(r0, nr)]))
    pltpu.sync_copy(ones, ones_v)
    plsc.subcore_barrier()

    def issue(j, b):
        pltpu.async_copy(ones_v, deg_s.at[didx.at[j]], dsems[b], add=True)

    def wait(b):
        pltpu.make_async_copy(ones_v, deg_s.at[pl.ds(0, CH)],
                              dsems[b]).wait()

    for b in range(NSLOT):
        issue(b, b)

    def body(q, carry):
        for b in range(NSLOT):
            wait(b)
            issue(q * NSLOT + b, b)
        return carry

    lax.fori_loop(1, CPW // NSLOT, body, 0)
    for b in range(NSLOT):
        wait(b)
    plsc.subcore_barrier()
    _per_tile_rows(s, lambda r0, nr: pltpu.sync_copy(
        deg_s.at[pl.ds(r0, nr)],
        deg2.at[pl.ds(pl.multiple_of(c * N + r0, 8), nr)]))


def _edge_gather_body(pq2, e2, s_out,
                      sidx, didx, a0, a1, a2, a3, b0, b1, b2, b3,
                      ga0, ga1, ga2, ga3, gb0, gb1, gb2, gb3,
                      t0, t1, t2, t3):
    c = lax.axis_index("c")
    s = lax.axis_index("s")
    w = s * 2 + c
    rows_a = (a0, a1, a2, a3)
    rows_b = (b0, b1, b2, b3)
    gsa = (ga0, ga1, ga2, ga3)
    gsb = (gb0, gb1, gb2, gb3)
    sts = (t0, t1, t2, t3)
    pltpu.sync_copy(e2.at[pl.ds(w * CPW, CPW)], sidx)
    pltpu.sync_copy(e2.at[pl.ds(NCHP + w * CPW, CPW)], didx)
    base = w * CPW

    def issue_a(j, b):
        pltpu.async_copy(pq2.at[sidx.at[j]], rows_a[b], gsa[b])

    def issue_b(j, b):
        pltpu.async_copy(pq2.at[didx.at[j]], rows_b[b], gsb[b])

    nq = CPW // NSLOT

    def process_quad(q, reissue):
        for b in range(NSLOT):
            j = q * NSLOT + b
            g = base + j
            pltpu.make_async_copy(pq2.at[pl.ds(0, CH)], rows_a[b],
                                  gsa[b]).wait()
            pltpu.make_async_copy(pq2.at[pl.ds(0, CH)], rows_b[b],
                                  gsb[b]).wait()

            def add_row(r, carry, _b=b):
                for k in range(H // 16):
                    sl = pl.ds(k * 16, 16)
                    rows_b[_b][r, sl] = rows_b[_b][r, sl] + rows_a[_b][r, sl]
                return carry

            lax.fori_loop(0, CH, add_row, 0)
            if reissue:
                issue_a(j + NSLOT, b)

            @pl.when(g < NCHUNK)
            def _():
                pltpu.async_copy(
                    rows_b[b],
                    s_out.at[pl.ds(pl.multiple_of(g * CH, 8), CH)], sts[b])
        if reissue:
            for b in range(NSLOT):
                j = q * NSLOT + b
                g = base + j

                @pl.when(g < NCHUNK)
                def _():
                    pltpu.make_async_copy(rows_b[b], s_out.at[pl.ds(0, CH)],
                                          sts[b]).wait()
                issue_b(j + NSLOT, b)

    for b in range(NSLOT):
        issue_a(b, b)
        issue_b(b, b)

    def body(q, carry):
        process_quad(q, True)
        return carry

    lax.fori_loop(0, nq - 1, body, 0)
    process_quad(nq - 1, False)
    for b in range(NSLOT):
        g = base + (nq - 1) * NSLOT + b

        @pl.when(g < NCHUNK)
        def _():
            pltpu.make_async_copy(rows_b[b], s_out.at[pl.ds(0, CH)],
                                  sts[b]).wait()


@functools.cache
def _sc_calls():
    mesh = plsc.VectorSubcoreMesh(core_axis_name="c", subcore_axis_name="s")
    params = pltpu.CompilerParams(use_tc_tiling_on_sc=False)
    idx_scratch = [pltpu.VMEM((CH,), jnp.int32)] * (2 * NSLOT)
    rows_scratch = [pltpu.VMEM((CH, HH), _f32)] * NSLOT
    sems = [pltpu.SemaphoreType.DMA] * NSLOT
    agg = pl.kernel(
        _agg_body,
        compiler_params=params,
        out_type=jax.ShapeDtypeStruct((2 * N, HH), _f32),
        mesh=mesh,
        scratch_types=[
            pltpu.VMEM_SHARED((N + 8, HH), _f32),
            *idx_scratch, *idx_scratch,
            *rows_scratch,
            *sems, *sems, *sems, *sems,
        ],
    )
    deg = pl.kernel(
        _deg_body,
        compiler_params=params,
        out_type=jax.ShapeDtypeStruct((2 * N, WD), _f32),
        mesh=mesh,
        scratch_types=[
            pltpu.VMEM_SHARED((N + 8, WD), _f32),
            pltpu.VMEM((CPW, CH), jnp.int32),
            pltpu.VMEM((CH, WD), _f32),
            *sems,
        ],
    )
    edge_gather = pl.kernel(
        _edge_gather_body,
        compiler_params=params,
        out_type=jax.ShapeDtypeStruct((E, H), _f32),
        mesh=mesh,
        scratch_types=[
            pltpu.VMEM((CPW, CH), jnp.int32),
            pltpu.VMEM((CPW, CH), jnp.int32),
            *([pltpu.VMEM((CH, H), _f32)] * (2 * NSLOT)),
            *sems, *sems, *sems,
        ],
    )
    return agg, deg, edge_gather


_BN = 1000
_BE = 2000


def _proj_body(x_ref, wpt_ref, bp_ref, out_ref):
    h = jnp.dot(x_ref[...], wpt_ref[...], preferred_element_type=_f32)
    h = h + bp_ref[...]
    out_ref[0] = h[:, :HH]
    out_ref[1] = h[:, HH:]


def _sage_update(agg_a, agg_b, deg_a, deg_b, h_a, h_b, wlt, bl, wrt, g, be):
    agg = jnp.concatenate([agg_a[...], agg_b[...]], axis=1)
    d = jnp.maximum(deg_a[...][:, :1] + deg_b[...][:, :1], 1.0)
    mean = agg / d
    h = jnp.concatenate([h_a[...], h_b[...]], axis=1)
    t = (jnp.dot(mean, wlt[...], preferred_element_type=_f32) + bl[...]
         + jnp.dot(h, wrt[...], preferred_element_type=_f32))
    mu = jnp.mean(t, axis=1, keepdims=True)
    var = jnp.mean((t - mu) * (t - mu), axis=1, keepdims=True)
    y = (t - mu) * lax.rsqrt(var + 1e-5) * g[...] + be[...]
    return jnp.maximum(y, 0.0)


def _update_body(agg_a, agg_b, deg_a, deg_b, h_a, h_b, wlt, bl, wrt, g, be,
                 out_ref):
    y = _sage_update(agg_a, agg_b, deg_a, deg_b, h_a, h_b, wlt, bl, wrt, g,
                     be)
    out_ref[0] = y[:, :HH]
    out_ref[1] = y[:, HH:]


def _update_pq_body(agg_a, agg_b, deg_a, deg_b, h_a, h_b, wlt, bl, wrt, g, be,
                    w1at, w1bt, b1, out_ref):
    y = _sage_update(agg_a, agg_b, deg_a, deg_b, h_a, h_b, wlt, bl, wrt, g,
                     be)
    out_ref[0] = jnp.dot(y, w1at[...], preferred_element_type=_f32)
    out_ref[1] = jnp.dot(y, w1bt[...], preferred_element_type=_f32) + b1[...]


def _softplus(x):
    return jnp.maximum(x, 0.0) + jnp.log(1.0 + jnp.exp(-jnp.abs(x)))


def _edge_mlp_body(s2_ref, ea_lo, ea_hi, w1ct, w2t2, b2_2, w3t2, b3, out_ref):
    lo = jnp.dot(ea_lo[...], w1ct[...], preferred_element_type=_f32)
    hi = jnp.dot(ea_hi[...], w1ct[...], preferred_element_type=_f32)
    z1 = jnp.maximum(s2_ref[...] + jnp.concatenate([lo, hi], axis=1), 0.0)
    z2 = jnp.maximum(jnp.dot(z1, w2t2[...], preferred_element_type=_f32)
                     + b2_2[...], 0.0)
    p = jnp.dot(z2, w3t2[...], preferred_element_type=_f32) + b3[0, 0]
    out_ref[...] = _softplus(p.T)


def _halved(i):
    return (i, 0)


def _halved_hi(i):
    return (N // _BN + i, 0)


def _full(i):
    return (0, 0)


_h2_spec = [pl.BlockSpec((_BN, HH), _halved), pl.BlockSpec((_BN, HH), _halved_hi)]
_w64_spec = pl.BlockSpec((H, H), _full)
_row64_spec = pl.BlockSpec((1, H), _full)


def _proj_call(x, wpt, bp2):
    out = pl.pallas_call(
        _proj_body,
        grid=(N // _BN,),
        in_specs=[pl.BlockSpec((_BN, 128), _halved), pl.BlockSpec((128, H), _full),
                  _row64_spec],
        out_specs=pl.BlockSpec((2, _BN, HH), lambda i: (0, i, 0)),
        out_shape=jax.ShapeDtypeStruct((2, N, HH), _f32),
    )(x, wpt, bp2)
    return out.reshape(2 * N, HH)


_deg_spec = [pl.BlockSpec((_BN, WD), _halved), pl.BlockSpec((_BN, WD), _halved_hi)]


def _update_call(agg2, deg2, h2, wlt, bl2, wrt, g2, be2):
    out = pl.pallas_call(
        _update_body,
        grid=(N // _BN,),
        in_specs=_h2_spec + _deg_spec + _h2_spec
        + [_w64_spec, _row64_spec, _w64_spec, _row64_spec, _row64_spec],
        out_specs=pl.BlockSpec((2, _BN, HH), lambda i: (0, i, 0)),
        out_shape=jax.ShapeDtypeStruct((2, N, HH), _f32),
    )(agg2, agg2, deg2, deg2, h2, h2, wlt, bl2, wrt, g2, be2)
    return out.reshape(2 * N, HH)


def _update_pq_call(agg2, deg2, h2, wlt, bl2, wrt, g2, be2, w1at, w1bt, b12):
    out = pl.pallas_call(
        _update_pq_body,
        grid=(N // _BN,),
        in_specs=_h2_spec + _deg_spec + _h2_spec
        + [_w64_spec, _row64_spec, _w64_spec, _row64_spec, _row64_spec,
           _w64_spec, _w64_spec, _row64_spec],
        out_specs=pl.BlockSpec((2, _BN, H), lambda i: (0, i, 0)),
        out_shape=jax.ShapeDtypeStruct((2, N, H), _f32),
    )(agg2, agg2, deg2, deg2, h2, h2, wlt, bl2, wrt, g2, be2, w1at, w1bt, b12)
    return out.reshape(2 * N, H)


_BE2 = 3200


def _edge_mlp_call(s_arr, edge_attr, w1ct, w2t, b2, w3row, b32):
    s2d = s_arr.reshape(E // 2, 2 * H)
    w2t2 = jnp.zeros((2 * H, H), _f32)
    w2t2 = w2t2.at[:H, :HH].set(w2t).at[H:, HH:].set(w2t)
    b2_2 = jnp.concatenate([b2, b2]).reshape(1, H)
    w3t2 = jnp.zeros((H, 2), _f32)
    w3t2 = w3t2.at[:HH, 0].set(w3row).at[HH:, 1].set(w3row)
    ngrid = E // 2 // _BE2
    out = pl.pallas_call(
        _edge_mlp_body,
        grid=(ngrid,),
        in_specs=[pl.BlockSpec((_BE2, 2 * H), _halved),
                  pl.BlockSpec((_BE2, 16), _halved),
                  pl.BlockSpec((_BE2, 16), lambda i: (ngrid + i, 0)),
                  pl.BlockSpec((16, H), _full),
                  pl.BlockSpec((2 * H, H), _full),
                  pl.BlockSpec((1, H), _full),
                  pl.BlockSpec((H, 2), _full),
                  pl.BlockSpec((1, 1), _full)],
        out_specs=pl.BlockSpec((2, _BE2), lambda i: (0, i)),
        out_shape=jax.ShapeDtypeStruct((2, E // 2), _f32),
    )(s2d, edge_attr, edge_attr, w1ct, w2t2, b2_2, w3t2, b32)
    return out.reshape(E)


def kernel(x, edge_index, edge_attr, Wp, bp,
           Wl0, bl0, Wr0, g0, be0,
           Wl1, bl1, Wr1, g1, be1,
           Wl2, bl2, Wr2, g2, be2,
           W1, b1, W2, b2, W3, b3):
    src = edge_index[0]
    dst = edge_index[1]
    pad0 = jnp.zeros((EP - E,), jnp.int32)
    padn = jnp.full((EP - E,), N, jnp.int32)
    src_a = jnp.concatenate([src, pad0]).reshape(NCHP, CH)
    src_b = jnp.concatenate([src + N, padn]).reshape(NCHP, CH)
    dstp = jnp.concatenate([dst, padn]).reshape(NCHP, CH)
    s2 = jnp.concatenate([src_a, src_b], 0)
    dn = dst + N
    j = jnp.arange(E, dtype=jnp.int32)
    perm = (j >> 1) + (j & 1) * (E // 2)
    src_p = src[perm]
    dn_p = dn[perm]
    e2 = jnp.concatenate([
        jnp.concatenate([src_p, pad0]).reshape(NCHP, CH),
        jnp.concatenate([dn_p, padn]).reshape(NCHP, CH)], 0)
    z = jnp.zeros((N, HH), _f32)
    zd = jnp.zeros((N, WD), _f32)
    ones = jnp.ones((CH, WD), _f32)

    agg_call, deg_call, edge_gather_call = _sc_calls()

    h2 = _proj_call(x, Wp.T, bp.reshape(1, H))
    deg2 = deg_call(dstp, zd, ones)

    agg2 = agg_call(h2, s2, dstp, z)
    h2 = _update_call(agg2, deg2, h2, Wl0.T, bl0.reshape(1, H), Wr0.T,
                      g0.reshape(1, H), be0.reshape(1, H))

    agg2 = agg_call(h2, s2, dstp, z)
    h2 = _update_call(agg2, deg2, h2, Wl1.T, bl1.reshape(1, H), Wr1.T,
                      g1.reshape(1, H), be1.reshape(1, H))

    agg2 = agg_call(h2, s2, dstp, z)
    w1at = W1[:, :H].T
    w1bt = W1[:, H:2 * H].T
    pq2 = _update_pq_call(agg2, deg2, h2, Wl2.T, bl2.reshape(1, H), Wr2.T,
                          g2.reshape(1, H), be2.reshape(1, H),
                          w1at, w1bt, b1.reshape(1, H))

    s_arr = edge_gather_call(pq2, e2)

    return _edge_mlp_call(s_arr, edge_attr, W1[:, 2 * H:].T, W2.T,
                          b2, W3[0], b3.reshape(1, 1))


# speedup vs baseline: 1.4631x; 1.0669x over previous
import functools

import jax
import jax.numpy as jnp
from jax import lax
from jax.experimental import pallas as pl
from jax.experimental.pallas import tpu as pltpu
from jax.experimental.pallas import tpu_sc as plsc

N = 50000
E = 800000
H = 64
HH = 32
CH = 128
NCHUNK = E // CH
NCHP = 6272
EP = NCHP * CH
CPT = NCHP // 16
CPW = NCHP // 32
NSLOT = 4
NTILES = 16
RSTEP = 3128
WD = 8

_f32 = jnp.float32


def _per_tile_rows(s, fn):
    r0 = pl.multiple_of(s * RSTEP, 8)

    @pl.when(s < NTILES - 1)
    def _():
        fn(r0, RSTEP)

    @pl.when(s == NTILES - 1)
    def _():
        fn((NTILES - 1) * RSTEP, N - (NTILES - 1) * RSTEP)


def _agg_body(h2, s2, dstp, z, agg2, agg_s,
              i0, i1, i2, i3, i4, i5, i6, i7,
              j0, j1, j2, j3, j4, j5, j6, j7,
              r0_, r1_, r2_, r3_,
              e0, e1, e2_, e3, e4, e5, e6, e7,
              g0, g1, g2, g3, s0, s1, s2_, s3):
    c = lax.axis_index("c")
    s = lax.axis_index("s")
    isrc = (i0, i1, i2, i3, i4, i5, i6, i7)
    idst = (j0, j1, j2, j3, j4, j5, j6, j7)
    rows = (r0_, r1_, r2_, r3_)
    isems = (e0, e1, e2_, e3, e4, e5, e6, e7)
    gsems = (g0, g1, g2, g3)
    ssems = (s0, s1, s2_, s3)
    _per_tile_rows(s, lambda r0, nr: pltpu.sync_copy(
        z.at[pl.ds(r0, nr)], agg_s.at[pl.ds(r0, nr)]))
    plsc.subcore_barrier()

    base = s * CPT

    def issue_idx(i, a):
        pltpu.async_copy(s2.at[c * NCHP + base + i], isrc[a], isems[a])
        pltpu.async_copy(dstp.at[base + i], idst[a], isems[a])

    def wait_idx(a):
        pltpu.make_async_copy(s2.at[0], isrc[a], isems[a]).wait()
        pltpu.make_async_copy(dstp.at[0], idst[a], isems[a]).wait()

    def issue_gather(a, b):
        pltpu.async_copy(h2.at[isrc[a]], rows[b], gsems[b])

    def wait_gather(b):
        pltpu.make_async_copy(h2.at[pl.ds(0, CH)], rows[b], gsems[b]).wait()

    def issue_scatter(a, b):
        pltpu.async_copy(rows[b], agg_s.at[idst[a]], ssems[b], add=True)

    def wait_scatter(b):
        pltpu.make_async_copy(rows[b], agg_s.at[pl.ds(0, CH)],
                              ssems[b]).wait()

    for a in range(2 * NSLOT):
        issue_idx(a, a)
    for b in range(NSLOT):
        wait_idx(b)
        issue_gather(b, b)

    def do_block(p, last):
        for h in range(2):
            for b in range(NSLOT):
                a = 4 * h + b
                i = 8 * p + a
                wait_gather(b)
                issue_scatter(a, b)
            for b in range(NSLOT):
                a = 4 * h + b
                i = 8 * p + a
                if last and h == 1:
                    wait_scatter(b)
                    continue
                wait_scatter(b)
                if not (last and h == 0):
                    issue_idx(i + 8, a)
                wait_idx(4 * (1 - h) + b)
                issue_gather(4 * (1 - h) + b, b)

    def body(p, carry):
        do_block(p, False)
        return carry

    nblk = CPT // (2 * NSLOT)
    lax.fori_loop(0, nblk - 1, body, 0)
    do_block(nblk - 1, True)

    plsc.subcore_barrier()
    _per_tile_rows(s, lambda r0, nr: pltpu.sync_copy(
        agg_s.at[pl.ds(r0, nr)],
        agg2.at[pl.ds(pl.multiple_of(c * N + r0, 8), nr)]))


def _deg_body(dstp, zd, ones, deg2, deg_s, didx, ones_v, d0, d1, d2, d3):
    c = lax.axis_index("c")
    s = lax.axis_index("s")
    dsems = (d0, d1, d2, d3)
    base = (c * 16 + s) * CPW
    pltpu.sync_copy(dstp.at[pl.ds(base, CPW)], didx)
    _per_tile_rows(s, lambda r0, nr: pltpu.sync_copy(
        zd.at[pl.ds(r0, nr)], deg_s.at[pl.ds(r0, nr)]))
    pltpu.sync_copy(ones, ones_v)
    plsc.subcore_barrier()

    def issue(j, b):
        pltpu.async_copy(ones_v, deg_s.at[didx.at[j]], dsems[b], add=True)

    def wait(b):
        pltpu.make_async_copy(ones_v, deg_s.at[pl.ds(0, CH)],
                              dsems[b]).wait()

    for b in range(NSLOT):
        issue(b, b)

    def body(q, carry):
        for b in range(NSLOT):
            wait(b)
            issue(q * NSLOT + b, b)
        return carry

    lax.fori_loop(1, CPW // NSLOT, body, 0)
    for b in range(NSLOT):
        wait(b)
    plsc.subcore_barrier()
    _per_tile_rows(s, lambda r0, nr: pltpu.sync_copy(
        deg_s.at[pl.ds(r0, nr)],
        deg2.at[pl.ds(pl.multiple_of(c * N + r0, 8), nr)]))


def _edge_gather_body(pq2, e2, s_out,
                      sidx, didx, a0, a1, a2, a3, b0, b1, b2, b3,
                      ga0, ga1, ga2, ga3, gb0, gb1, gb2, gb3,
                      t0, t1, t2, t3):
    c = lax.axis_index("c")
    s = lax.axis_index("s")
    w = s * 2 + c
    rows_a = (a0, a1, a2, a3)
    rows_b = (b0, b1, b2, b3)
    gsa = (ga0, ga1, ga2, ga3)
    gsb = (gb0, gb1, gb2, gb3)
    sts = (t0, t1, t2, t3)
    pltpu.sync_copy(e2.at[pl.ds(w * CPW, CPW)], sidx)
    pltpu.sync_copy(e2.at[pl.ds(NCHP + w * CPW, CPW)], didx)
    base = w * CPW

    def issue_a(j, b):
        pltpu.async_copy(pq2.at[sidx.at[j]], rows_a[b], gsa[b])

    def issue_b(j, b):
        pltpu.async_copy(pq2.at[didx.at[j]], rows_b[b], gsb[b])

    nq = CPW // NSLOT

    def process_quad(q, reissue):
        for b in range(NSLOT):
            j = q * NSLOT + b
            g = base + j
            pltpu.make_async_copy(pq2.at[pl.ds(0, CH)], rows_a[b],
                                  gsa[b]).wait()
            pltpu.make_async_copy(pq2.at[pl.ds(0, CH)], rows_b[b],
                                  gsb[b]).wait()

            def add_row(r, carry, _b=b):
                for k in range(H // 16):
                    sl = pl.ds(k * 16, 16)
                    rows_b[_b][r, sl] = rows_b[_b][r, sl] + rows_a[_b][r, sl]
                return carry

            lax.fori_loop(0, CH, add_row, 0)
            if reissue:
                issue_a(j + NSLOT, b)

            @pl.when(g < NCHUNK)
            def _():
                pltpu.async_copy(
                    rows_b[b],
                    s_out.at[pl.ds(pl.multiple_of(g * CH, 8), CH)], sts[b])
        if reissue:
            for b in range(NSLOT):
                j = q * NSLOT + b
                g = base + j

                @pl.when(g < NCHUNK)
                def _():
                    pltpu.make_async_copy(rows_b[b], s_out.at[pl.ds(0, CH)],
                                          sts[b]).wait()
                issue_b(j + NSLOT, b)

    for b in range(NSLOT):
        issue_a(b, b)
        issue_b(b, b)

    def body(q, carry):
        process_quad(q, True)
        return carry

    lax.fori_loop(0, nq - 1, body, 0)
    process_quad(nq - 1, False)
    for b in range(NSLOT):
        g = base + (nq - 1) * NSLOT + b

        @pl.when(g < NCHUNK)
        def _():
            pltpu.make_async_copy(rows_b[b], s_out.at[pl.ds(0, CH)],
                                  sts[b]).wait()


@functools.cache
def _sc_calls():
    mesh = plsc.VectorSubcoreMesh(core_axis_name="c", subcore_axis_name="s")
    params = pltpu.CompilerParams(use_tc_tiling_on_sc=False)
    idx_scratch = [pltpu.VMEM((CH,), jnp.int32)] * (2 * NSLOT)
    rows_scratch = [pltpu.VMEM((CH, HH), _f32)] * NSLOT
    sems = [pltpu.SemaphoreType.DMA] * NSLOT
    agg = pl.kernel(
        _agg_body,
        compiler_params=params,
        out_type=jax.ShapeDtypeStruct((2 * N, HH), _f32),
        mesh=mesh,
        scratch_types=[
            pltpu.VMEM_SHARED((N + 8, HH), _f32),
            *idx_scratch, *idx_scratch,
            *rows_scratch,
            *sems, *sems, *sems, *sems,
        ],
    )
    deg = pl.kernel(
        _deg_body,
        compiler_params=params,
        out_type=jax.ShapeDtypeStruct((2 * N, WD), _f32),
        mesh=mesh,
        scratch_types=[
            pltpu.VMEM_SHARED((N + 8, WD), _f32),
            pltpu.VMEM((CPW, CH), jnp.int32),
            pltpu.VMEM((CH, WD), _f32),
            *sems,
        ],
    )
    edge_gather = pl.kernel(
        _edge_gather_body,
        compiler_params=params,
        out_type=jax.ShapeDtypeStruct((E, H), _f32),
        mesh=mesh,
        scratch_types=[
            pltpu.VMEM((CPW, CH), jnp.int32),
            pltpu.VMEM((CPW, CH), jnp.int32),
            *([pltpu.VMEM((CH, H), _f32)] * (2 * NSLOT)),
            *sems, *sems, *sems,
        ],
    )
    return agg, deg, edge_gather


_BN = 1000
_BE = 2000


def _proj_body(x_ref, wpt_ref, bp_ref, out_ref):
    h = jnp.dot(x_ref[...], wpt_ref[...], preferred_element_type=_f32)
    h = h + bp_ref[...]
    out_ref[0] = h[:, :HH]
    out_ref[1] = h[:, HH:]


def _sage_update(agg_a, agg_b, deg_a, deg_b, h_a, h_b, wlt, bl, wrt, g, be):
    agg = jnp.concatenate([agg_a[...], agg_b[...]], axis=1)
    d = jnp.maximum(deg_a[...][:, :1] + deg_b[...][:, :1], 1.0)
    mean = agg / d
    h = jnp.concatenate([h_a[...], h_b[...]], axis=1)
    t = (jnp.dot(mean, wlt[...], preferred_element_type=_f32) + bl[...]
         + jnp.dot(h, wrt[...], preferred_element_type=_f32))
    mu = jnp.mean(t, axis=1, keepdims=True)
    var = jnp.mean((t - mu) * (t - mu), axis=1, keepdims=True)
    y = (t - mu) * lax.rsqrt(var + 1e-5) * g[...] + be[...]
    return jnp.maximum(y, 0.0)


def _update_body(agg_a, agg_b, deg_a, deg_b, h_a, h_b, wlt, bl, wrt, g, be,
                 out_ref):
    y = _sage_update(agg_a, agg_b, deg_a, deg_b, h_a, h_b, wlt, bl, wrt, g,
                     be)
    out_ref[0] = y[:, :HH]
    out_ref[1] = y[:, HH:]


def _update_pq_body(agg_a, agg_b, deg_a, deg_b, h_a, h_b, wlt, bl, wrt, g, be,
                    w1at, w1bt, b1, out_ref):
    y = _sage_update(agg_a, agg_b, deg_a, deg_b, h_a, h_b, wlt, bl, wrt, g,
                     be)
    out_ref[0] = jnp.dot(y, w1at[...], preferred_element_type=_f32)
    out_ref[1] = jnp.dot(y, w1bt[...], preferred_element_type=_f32) + b1[...]


def _softplus(x):
    return jnp.maximum(x, 0.0) + jnp.log(1.0 + jnp.exp(-jnp.abs(x)))


def _edge_mlp_body(s2_ref, ea_lo, ea_hi, w1ct, w2t2, b2_2, w3t2, b3, out_ref):
    dn = (((0,), (0,)), ((), ()))
    lo = lax.dot_general(ea_lo[...], w1ct[...], dn,
                         preferred_element_type=_f32)
    hi = lax.dot_general(ea_hi[...], w1ct[...], dn,
                         preferred_element_type=_f32)
    z1 = jnp.maximum(s2_ref[...] + jnp.concatenate([lo, hi], axis=1), 0.0)
    z2 = jnp.maximum(jnp.dot(z1, w2t2[...], preferred_element_type=_f32)
                     + b2_2[...], 0.0)
    p = jnp.dot(z2, w3t2[...], preferred_element_type=_f32) + b3[0, 0]
    out_ref[...] = _softplus(p.T)


def _halved(i):
    return (i, 0)


def _halved_hi(i):
    return (N // _BN + i, 0)


def _full(i):
    return (0, 0)


_h2_spec = [pl.BlockSpec((_BN, HH), _halved), pl.BlockSpec((_BN, HH), _halved_hi)]
_w64_spec = pl.BlockSpec((H, H), _full)
_row64_spec = pl.BlockSpec((1, H), _full)


def _proj_call(x, wpt, bp2):
    out = pl.pallas_call(
        _proj_body,
        grid=(N // _BN,),
        in_specs=[pl.BlockSpec((_BN, 128), _halved), pl.BlockSpec((128, H), _full),
                  _row64_spec],
        out_specs=pl.BlockSpec((2, _BN, HH), lambda i: (0, i, 0)),
        out_shape=jax.ShapeDtypeStruct((2, N, HH), _f32),
    )(x, wpt, bp2)
    return out.reshape(2 * N, HH)


_deg_spec = [pl.BlockSpec((_BN, WD), _halved), pl.BlockSpec((_BN, WD), _halved_hi)]


def _update_call(agg2, deg2, h2, wlt, bl2, wrt, g2, be2):
    out = pl.pallas_call(
        _update_body,
        grid=(N // _BN,),
        in_specs=_h2_spec + _deg_spec + _h2_spec
        + [_w64_spec, _row64_spec, _w64_spec, _row64_spec, _row64_spec],
        out_specs=pl.BlockSpec((2, _BN, HH), lambda i: (0, i, 0)),
        out_shape=jax.ShapeDtypeStruct((2, N, HH), _f32),
    )(agg2, agg2, deg2, deg2, h2, h2, wlt, bl2, wrt, g2, be2)
    return out.reshape(2 * N, HH)


def _update_pq_call(agg2, deg2, h2, wlt, bl2, wrt, g2, be2, w1at, w1bt, b12):
    out = pl.pallas_call(
        _update_pq_body,
        grid=(N // _BN,),
        in_specs=_h2_spec + _deg_spec + _h2_spec
        + [_w64_spec, _row64_spec, _w64_spec, _row64_spec, _row64_spec,
           _w64_spec, _w64_spec, _row64_spec],
        out_specs=pl.BlockSpec((2, _BN, H), lambda i: (0, i, 0)),
        out_shape=jax.ShapeDtypeStruct((2, N, H), _f32),
    )(agg2, agg2, deg2, deg2, h2, h2, wlt, bl2, wrt, g2, be2, w1at, w1bt, b12)
    return out.reshape(2 * N, H)


_BE2 = 3200


def _edge_mlp_call(s_arr, edge_attr, w1ct, w2t, b2, w3row, b32):
    s2d = s_arr.reshape(E // 2, 2 * H)
    w2t2 = jnp.zeros((2 * H, H), _f32)
    w2t2 = w2t2.at[:H, :HH].set(w2t).at[H:, HH:].set(w2t)
    b2_2 = jnp.concatenate([b2, b2]).reshape(1, H)
    w3t2 = jnp.zeros((H, 2), _f32)
    w3t2 = w3t2.at[:HH, 0].set(w3row).at[HH:, 1].set(w3row)
    ngrid = E // 2 // _BE2
    eat = edge_attr.T
    out = pl.pallas_call(
        _edge_mlp_body,
        grid=(ngrid,),
        in_specs=[pl.BlockSpec((_BE2, 2 * H), _halved),
                  pl.BlockSpec((16, _BE2), lambda i: (0, i)),
                  pl.BlockSpec((16, _BE2), lambda i: (0, ngrid + i)),
                  pl.BlockSpec((16, H), _full),
                  pl.BlockSpec((2 * H, H), _full),
                  pl.BlockSpec((1, H), _full),
                  pl.BlockSpec((H, 2), _full),
                  pl.BlockSpec((1, 1), _full)],
        out_specs=pl.BlockSpec((2, _BE2), lambda i: (0, i)),
        out_shape=jax.ShapeDtypeStruct((2, E // 2), _f32),
    )(s2d, eat, eat, w1ct, w2t2, b2_2, w3t2, b32)
    return out.reshape(E)


def kernel(x, edge_index, edge_attr, Wp, bp,
           Wl0, bl0, Wr0, g0, be0,
           Wl1, bl1, Wr1, g1, be1,
           Wl2, bl2, Wr2, g2, be2,
           W1, b1, W2, b2, W3, b3):
    src = edge_index[0]
    dst = edge_index[1]
    pad0 = jnp.zeros((EP - E,), jnp.int32)
    padn = jnp.full((EP - E,), N, jnp.int32)
    src_a = jnp.concatenate([src, pad0]).reshape(NCHP, CH)
    src_b = jnp.concatenate([src + N, padn]).reshape(NCHP, CH)
    dstp = jnp.concatenate([dst, padn]).reshape(NCHP, CH)
    s2 = jnp.concatenate([src_a, src_b], 0)
    dn = dst + N
    j = jnp.arange(E, dtype=jnp.int32)
    perm = (j >> 1) + (j & 1) * (E // 2)
    src_p = src[perm]
    dn_p = dn[perm]
    e2 = jnp.concatenate([
        jnp.concatenate([src_p, pad0]).reshape(NCHP, CH),
        jnp.concatenate([dn_p, padn]).reshape(NCHP, CH)], 0)
    z = jnp.zeros((N, HH), _f32)
    zd = jnp.zeros((N, WD), _f32)
    ones = jnp.ones((CH, WD), _f32)

    agg_call, deg_call, edge_gather_call = _sc_calls()

    h2 = _proj_call(x, Wp.T, bp.reshape(1, H))
    deg2 = deg_call(dstp, zd, ones)

    agg2 = agg_call(h2, s2, dstp, z)
    h2 = _update_call(agg2, deg2, h2, Wl0.T, bl0.reshape(1, H), Wr0.T,
                      g0.reshape(1, H), be0.reshape(1, H))

    agg2 = agg_call(h2, s2, dstp, z)
    h2 = _update_call(agg2, deg2, h2, Wl1.T, bl1.reshape(1, H), Wr1.T,
                      g1.reshape(1, H), be1.reshape(1, H))

    agg2 = agg_call(h2, s2, dstp, z)
    w1at = W1[:, :H].T
    w1bt = W1[:, H:2 * H].T
    pq2 = _update_pq_call(agg2, deg2, h2, Wl2.T, bl2.reshape(1, H), Wr2.T,
                          g2.reshape(1, H), be2.reshape(1, H),
                          w1at, w1bt, b1.reshape(1, H))

    s_arr = edge_gather_call(pq2, e2)

    return _edge_mlp_call(s_arr, edge_attr, W1[:, 2 * H:].T, W2.T,
                          b2, W3[0], b3.reshape(1, 1))
